# baseline (speedup 1.0000x reference)
import functools
import math

import jax
import jax.numpy as jnp
from jax import lax
from jax.experimental import pallas as pl
from jax.experimental.pallas import tpu as pltpu
from jax.experimental.pallas import tpu_sc as plsc

N = 50000
E = 800000
EMBED = 64
HEADS = 8
DH = EMBED // HEADS

NB = 2000
EB = 4000
RROW = 16
QROW = 32
XROW = 16

F32 = jnp.float32
_PREC = lax.Precision.DEFAULT


def _ln(x, g, b, eps=1e-5):
    m = jnp.mean(x, axis=-1, keepdims=True)
    v = jnp.mean((x - m) * (x - m), axis=-1, keepdims=True)
    return (x - m) * lax.rsqrt(v + eps) * g + b


def _jmat(width, seg):
    r = lax.broadcasted_iota(jnp.int32, (width, width), 0) // seg
    c = lax.broadcasted_iota(jnp.int32, (width, width), 1) // seg
    return (r == c).astype(F32) * (1.0 / seg)


def _ln_mxu(x, g, b, jm, eps=1e-5):
    c0 = x - _dot(x, jm)
    v = _dot(c0 * c0, jm)
    return c0 * lax.rsqrt(v + eps) * g + b


def _groups_mats(width, seg):
    p = width // seg
    r = lax.broadcasted_iota(jnp.int32, (width, p), 0) // seg
    c = lax.broadcasted_iota(jnp.int32, (width, p), 1)
    avg = (r == c).astype(F32) * (1.0 / seg)
    return avg, (avg * seg).T


def _lnc(xc, g, b, width, seg, eps=1e-5):
    avg, exp_m = _groups_mats(width, seg)
    v = _dot(xc * xc, avg)
    return xc * _dot(lax.rsqrt(v + eps), exp_m) * g + b


def _dot(a, b):
    return jax.lax.dot_general(a, b, (((1,), (0,)), ((), ())),
                               precision=_PREC, preferred_element_type=F32)


def _head_mat():
    r = lax.broadcasted_iota(jnp.int32, (EMBED, HEADS), 0) // DH
    c = lax.broadcasted_iota(jnp.int32, (EMBED, HEADS), 1)
    return (r == c).astype(F32)


def _p1_body(x_ref, rot_ref, bosf_ref, bosrow_ref, w2_ref, w64_ref, vec_ref,
             center_ref, h_ref, q_ref):
    x0 = x_ref[:, 0:1]
    x1 = x_ref[:, 1:2]
    rot = rot_ref[...]
    r00 = rot[:, 0:1]
    r01 = rot[:, 1:2]
    r10 = rot[:, 2:3]
    r11 = rot[:, 3:4]
    rx0 = x0 * r00 + x1 * r10
    rx1 = x0 * r01 + x1 * r11
    w2 = w2_ref[...]
    j1 = _jmat(EMBED, EMBED)
    h1 = rx0 * w2[0:1, :] + rx1 * w2[1:2, :] + vec_ref[0]
    h1 = jnp.maximum(_ln_mxu(h1, vec_ref[1], vec_ref[2], j1), 0.0)
    h2 = _dot(h1, w64_ref[0]) + vec_ref[3]
    h2 = jnp.maximum(_ln_mxu(h2, vec_ref[4], vec_ref[5], j1), 0.0)
    c0 = _ln_mxu(_dot(h2, w64_ref[1]) + vec_ref[6], vec_ref[7], vec_ref[8], j1)
    bosf = bosf_ref[:, 0:1]
    center = c0 * (1.0 - bosf) + bosrow_ref[...] * bosf
    h = _ln_mxu(center, vec_ref[9], vec_ref[10], j1)
    q = _dot(h, w64_ref[2]) + vec_ref[11]
    center_ref[...] = center
    h_ref[...] = h
    q_ref[...] = q


def _run_p1(x, rot4, bosf, bosrow, w2, w64, vec):
    grid = (N // NB,)
    return pl.pallas_call(
        _p1_body,
        grid=grid,
        in_specs=[
            pl.BlockSpec((NB, 2), lambda i: (i, 0)),
            pl.BlockSpec((NB, 4), lambda i: (i, 0)),
            pl.BlockSpec((NB, 1), lambda i: (i, 0)),
            pl.BlockSpec((1, EMBED), lambda i: (0, 0)),
            pl.BlockSpec((2, EMBED), lambda i: (0, 0)),
            pl.BlockSpec((3, EMBED, EMBED), lambda i: (0, 0, 0)),
            pl.BlockSpec((12, EMBED), lambda i: (0, 0)),
        ],
        out_specs=[
            pl.BlockSpec((NB, EMBED), lambda i: (i, 0)),
            pl.BlockSpec((NB, EMBED), lambda i: (i, 0)),
            pl.BlockSpec((NB, EMBED), lambda i: (i, 0)),
        ],
        out_shape=[
            jax.ShapeDtypeStruct((N, EMBED), F32),
            jax.ShapeDtypeStruct((N, EMBED), F32),
            jax.ShapeDtypeStruct((N, EMBED), F32),
        ],
    )(x, rot4, bosf, bosrow, w2, w64, vec)


_NSC = 2
_NSUB = 16

def _sc_mesh():
    return plsc.VectorSubcoreMesh(core_axis_name="c", subcore_axis_name="s",
                                  num_cores=_NSC, num_subcores=_NSUB)
_SC_PARAMS = pltpu.CompilerParams(use_tc_tiling_on_sc=False)
_NW = _NSC * _NSUB
_G_CH = 1000
_G_EPW = E // _NW
_G_NIT = _G_EPW // _G_CH


def _gather_body(tr_hbm, tq_hbm, tx_hbm, dst_hbm, src_hbm,
                 gr_hbm, gq_hbm, gs_hbm,
                 idxd_v, idxs_v, bufr, bufq, bufs, semr, semq, sems):
    c = lax.axis_index("c")
    s = lax.axis_index("s")
    base = (c * _NSUB + s) * _G_EPW

    def body(i, carry):
        e0 = base + i * _G_CH
        pltpu.sync_copy(dst_hbm.at[pl.ds(e0, _G_CH)], idxd_v)
        pltpu.sync_copy(src_hbm.at[pl.ds(e0, _G_CH)], idxs_v)
        cr = pltpu.async_copy(tr_hbm.at[idxd_v], bufr, semr)
        cq = pltpu.async_copy(tq_hbm.at[idxd_v], bufq, semq)
        cs = pltpu.async_copy(tx_hbm.at[idxs_v], bufs, sems)
        cr.wait()
        pltpu.sync_copy(bufr, gr_hbm.at[pl.ds(e0, _G_CH)])
        cq.wait()
        pltpu.sync_copy(bufq, gq_hbm.at[pl.ds(e0, _G_CH)])
        cs.wait()
        pltpu.sync_copy(bufs, gs_hbm.at[pl.ds(e0, _G_CH)])
        return carry

    lax.fori_loop(0, _G_NIT, body, 0)


def _run_gather(tr_tab, tq_tab, tx_tab, dst, src):
    f = functools.partial(
        pl.kernel,
        out_type=(jax.ShapeDtypeStruct((E, RROW), F32),
                  jax.ShapeDtypeStruct((E, QROW), F32),
                  jax.ShapeDtypeStruct((E, XROW), F32)),
        mesh=_sc_mesh(),
        compiler_params=_SC_PARAMS,
        scratch_types=[
            pltpu.VMEM((_G_CH,), jnp.int32),
            pltpu.VMEM((_G_CH,), jnp.int32),
            pltpu.VMEM((_G_CH, RROW), F32),
            pltpu.VMEM((_G_CH, QROW), F32),
            pltpu.VMEM((_G_CH, XROW), F32),
            pltpu.SemaphoreType.DMA,
            pltpu.SemaphoreType.DMA,
            pltpu.SemaphoreType.DMA,
        ],
    )(_gather_body)
    return f(tr_tab, tq_tab, tx_tab, dst, src)


def _p3_body(gr_ref, gq_ref, gs_ref, ea_ref, w2_ref, wcat_ref, wal_ref,
             wkv_ref, vec_ref, c01_ref, cae_ref):
    q = gq_ref[...].astype(F32)
    a = jnp.concatenate([gs_ref[:, 0:4], ea_ref[:, 0:4]], axis=1)
    prod = a * gr_ref[:, 0:8]
    catc = _dot(prod, w2_ref[...]) + vec_ref[2]
    cat = jnp.maximum(_lnc(catc, vec_ref[0], vec_ref[1], 2 * EMBED, EMBED),
                      0.0)
    smc = _dot(cat, wcat_ref[...]) + vec_ref[3, 0:EMBED]
    t1 = jnp.maximum(
        _lnc(smc, vec_ref[3, EMBED:], vec_ref[4, 0:EMBED], EMBED, EMBED), 0.0)
    alc = _dot(t1, wal_ref[...]) + vec_ref[4, EMBED:]
    nbr = _lnc(alc, vec_ref[5, 0:EMBED], vec_ref[5, EMBED:], EMBED, EMBED)
    kv = _dot(nbr, wkv_ref[...]) + vec_ref[6]
    k = kv[:, 0:EMBED]
    v = kv[:, EMBED:]
    hm = _head_mat()
    alpha = _dot(q * k, hm) * (1.0 / math.sqrt(float(DH)))
    ae = jnp.exp(alpha)
    wv = v * _dot(ae, hm.T)
    c01_ref[0] = wv[:, 0:32]
    c01_ref[1] = wv[:, 32:64]
    cae_ref[...] = jnp.concatenate([ae, jnp.zeros((EB, 8), F32)], axis=1)


def _run_p3(gr, gqb, gs, ea, w2, wcat, wal, wkv, vec):
    grid = (E // EB,)
    return pl.pallas_call(
        _p3_body,
        grid=grid,
        in_specs=[
            pl.BlockSpec((EB, RROW), lambda i: (i, 0)),
            pl.BlockSpec((EB, EMBED), lambda i: (i, 0)),
            pl.BlockSpec((EB, XROW), lambda i: (i, 0)),
            pl.BlockSpec((EB, 4), lambda i: (i, 0)),
            pl.BlockSpec((8, 2 * EMBED), lambda i: (0, 0)),
            pl.BlockSpec((2 * EMBED, EMBED), lambda i: (0, 0)),
            pl.BlockSpec((EMBED, EMBED), lambda i: (0, 0)),
            pl.BlockSpec((EMBED, 2 * EMBED), lambda i: (0, 0)),
            pl.BlockSpec((7, 2 * EMBED), lambda i: (0, 0)),
        ],
        out_specs=[
            pl.BlockSpec((2, EB, 32), lambda i: (0, i, 0)),
            pl.BlockSpec((EB, 16), lambda i: (i, 0)),
        ],
        out_shape=[
            jax.ShapeDtypeStruct((2, E, 32), F32),
            jax.ShapeDtypeStruct((E, 16), F32),
        ],
    )(gr, gqb, gs, ea, w2, wcat, wal, wkv, vec)


_S_CH = 400
_S_EPW = E // _NSUB
_S_NIT = _S_EPW // _S_CH
_NPS = N // _NSUB


def _scat_wv_body(c01_hbm, dst_hbm, z_hbm, out_hbm, idx_v, buf, acc, sem):
    c = lax.axis_index("c")
    s = lax.axis_index("s")
    pltpu.sync_copy(z_hbm.at[pl.ds(s * _NPS, _NPS)],
                    acc.at[pl.ds(s * _NPS, _NPS)])
    plsc.subcore_barrier()
    base = s * _S_EPW

    def body(i, carry):
        e0 = base + i * _S_CH
        pltpu.sync_copy(dst_hbm.at[pl.ds(e0, _S_CH)], idx_v)
        pltpu.sync_copy(c01_hbm.at[c, pl.ds(e0, _S_CH)], buf)
        pltpu.sync_copy(buf, acc.at[idx_v], add=True)
        return carry

    lax.fori_loop(0, _S_NIT, body, 0)
    plsc.subcore_barrier()
    pltpu.sync_copy(acc.at[pl.ds(s * _NPS, _NPS)],
                    out_hbm.at[c, pl.ds(s * _NPS, _NPS)])


def _run_scat_wv(c01, dst, z32):
    f = functools.partial(
        pl.kernel,
        out_type=jax.ShapeDtypeStruct((2, N, 32), F32),
        mesh=_sc_mesh(),
        compiler_params=_SC_PARAMS,
        scratch_types=[
            pltpu.VMEM((_S_CH,), jnp.int32),
            pltpu.VMEM((_S_CH, 32), F32),
            pltpu.VMEM_SHARED((N, 32), F32),
            pltpu.SemaphoreType.DMA,
        ],
    )(_scat_wv_body)
    return f(c01, dst, z32)


_A_CH = 1000
_A_EPW = (E // 2) // _NSUB
_A_NIT = _A_EPW // _A_CH


def _scat_ae_body(cae_hbm, dst_hbm, z_hbm, out_hbm, idx_v, buf, acc, sem):
    c = lax.axis_index("c")
    s = lax.axis_index("s")
    pltpu.sync_copy(z_hbm.at[pl.ds(s * _NPS, _NPS)],
                    acc.at[pl.ds(s * _NPS, _NPS)])
    plsc.subcore_barrier()
    base = c * (E // 2) + s * _A_EPW

    def body(i, carry):
        e0 = base + i * _A_CH
        pltpu.sync_copy(dst_hbm.at[pl.ds(e0, _A_CH)], idx_v)
        pltpu.sync_copy(cae_hbm.at[pl.ds(e0, _A_CH)], buf)
        pltpu.sync_copy(buf, acc.at[idx_v], add=True)
        return carry

    lax.fori_loop(0, _A_NIT, body, 0)
    plsc.subcore_barrier()
    pltpu.sync_copy(acc.at[pl.ds(s * _NPS, _NPS)],
                    out_hbm.at[c, pl.ds(s * _NPS, _NPS)])


def _run_scat_ae(cae, dst, z16):
    f = functools.partial(
        pl.kernel,
        out_type=jax.ShapeDtypeStruct((2, N, 16), F32),
        mesh=_sc_mesh(),
        compiler_params=_SC_PARAMS,
        scratch_types=[
            pltpu.VMEM((_A_CH,), jnp.int32),
            pltpu.VMEM((_A_CH, 16), F32),
            pltpu.VMEM_SHARED((N, 16), F32),
            pltpu.SemaphoreType.DMA,
        ],
    )(_scat_ae_body)
    return f(cae, dst, z16)


def _p5_body(u0_ref, u1_ref, d0_ref, d1_ref, h_ref, c_ref,
             w5_ref, wout_ref, wm1_ref, wm2_ref, vec_ref, out_ref):
    dsum = d0_ref[:, 0:HEADS] + d1_ref[:, 0:HEADS]
    u = jnp.concatenate([u0_ref[...], u1_ref[...]], axis=1)
    hm = _head_mat()
    agg = u * _dot(1.0 / (dsum + 1e-16), hm.T)
    h = h_ref[...]
    cath = jnp.concatenate([agg, h], axis=1)
    gs = _dot(cath, w5_ref[...]) + vec_ref[0, 0:2 * EMBED]
    gate = jax.nn.sigmoid(gs[:, 0:EMBED])
    selfh = gs[:, EMBED:]
    upd = agg + gate * (selfh - agg)
    c2 = c_ref[...] + _dot(upd, wout_ref[...]) + vec_ref[1, 0:EMBED]
    j1 = _jmat(EMBED, EMBED)
    h2 = _ln_mxu(c2, vec_ref[1, EMBED:2 * EMBED],
                 vec_ref[1, 2 * EMBED:3 * EMBED], j1)
    ff = _dot(jnp.maximum(_dot(h2, wm1_ref[...]) + vec_ref[2], 0.0),
              wm2_ref[...]) + vec_ref[1, 3 * EMBED:]
    out_ref[...] = c2 + ff


def _run_p5(u0, u1, d0, d1, h, center, w5, wout, wm1, wm2, vec):
    grid = (N // NB,)
    return pl.pallas_call(
        _p5_body,
        grid=grid,
        in_specs=[
            pl.BlockSpec((NB, 32), lambda i: (i, 0)),
            pl.BlockSpec((NB, 32), lambda i: (i, 0)),
            pl.BlockSpec((NB, 16), lambda i: (i, 0)),
            pl.BlockSpec((NB, 16), lambda i: (i, 0)),
            pl.BlockSpec((NB, EMBED), lambda i: (i, 0)),
            pl.BlockSpec((NB, EMBED), lambda i: (i, 0)),
            pl.BlockSpec((2 * EMBED, 2 * EMBED), lambda i: (0, 0)),
            pl.BlockSpec((EMBED, EMBED), lambda i: (0, 0)),
            pl.BlockSpec((EMBED, 4 * EMBED), lambda i: (0, 0)),
            pl.BlockSpec((4 * EMBED, EMBED), lambda i: (0, 0)),
            pl.BlockSpec((3, 4 * EMBED), lambda i: (0, 0)),
        ],
        out_specs=pl.BlockSpec((NB, EMBED), lambda i: (i, 0)),
        out_shape=jax.ShapeDtypeStruct((N, EMBED), F32),
    )(u0, u1, d0, d1, h, center, w5, wout, wm1, wm2, vec)


def kernel(x, t, edge_index, edge_attr, bos_mask, rotate_mat, params):
    p = params
    src = edge_index[0]
    dst = edge_index[1]
    rot4 = rotate_mat.reshape(N, 4)
    bosf = bos_mask.astype(F32).reshape(N, 1)
    bosrow = p["bos_token"][t].reshape(1, EMBED)

    w2_1 = p["ce_l1"]["W"]
    w64_1 = jnp.stack([p["ce_l2"]["W"], p["ce_l3"]["W"], p["lin_q"]["W"]])
    vec_1 = jnp.stack([
        p["ce_l1"]["b"], p["ce_n1"]["g"], p["ce_n1"]["b"],
        p["ce_l2"]["b"], p["ce_n2"]["g"], p["ce_n2"]["b"],
        p["ce_l3"]["b"], p["ce_n3"]["g"], p["ce_n3"]["b"],
        p["norm1"]["g"], p["norm1"]["b"], p["lin_q"]["b"],
    ])
    center, h, q_nodes = _run_p1(x, rot4, bosf, bosrow, w2_1, w64_1, vec_1)

    rdup = jnp.stack([rot4[:, 0], rot4[:, 2], rot4[:, 1], rot4[:, 3]], axis=1)
    tr_tab = jnp.concatenate([rdup, rdup, jnp.zeros((N, RROW - 8), F32)],
                             axis=1)
    tq_tab = lax.bitcast_convert_type(
        q_nodes.astype(jnp.bfloat16).reshape(N, QROW, 2), F32)
    tx_tab = jnp.concatenate([x, x, jnp.zeros((N, XROW - 4), F32)], axis=1)
    gr, gq, gs = _run_gather(tr_tab, tq_tab, tx_tab, dst, src)
    gqb = lax.bitcast_convert_type(gq, jnp.bfloat16).reshape(E, EMBED)

    wa = p["nb0_l1"]["W"]
    wb = p["nb1_l1"]["W"]
    z64 = jnp.zeros((EMBED,), F32)
    sw = jnp.stack([
        jnp.concatenate([wa[0], z64]), jnp.concatenate([wa[0], z64]),
        jnp.concatenate([wa[1], z64]), jnp.concatenate([wa[1], z64]),
        jnp.concatenate([z64, wb[0]]), jnp.concatenate([z64, wb[0]]),
        jnp.concatenate([z64, wb[1]]), jnp.concatenate([z64, wb[1]]),
    ])
    cmat = jnp.eye(EMBED, dtype=F32) - jnp.full((EMBED, EMBED), 1.0 / EMBED,
                                                F32)
    ctr = lambda w: w @ cmat
    wcat = ctr(jnp.concatenate([p["nb0_l2"]["W"], p["nb1_l2"]["W"]], axis=0))
    wal_c = ctr(p["nb_al"]["W"])
    wkv = jnp.concatenate([p["lin_k"]["W"], p["lin_v"]["W"]], axis=1)
    cc = lambda a, b: jnp.concatenate([a, b])
    cv = lambda v: v - jnp.mean(v)
    sw = jnp.concatenate([sw[:, 0:EMBED] @ cmat, sw[:, EMBED:] @ cmat],
                         axis=1)
    vec_3 = jnp.stack([
        cc(p["nb0_n1"]["g"], p["nb1_n1"]["g"]),
        cc(p["nb0_n1"]["b"], p["nb1_n1"]["b"]),
        cc(cv(p["nb0_l1"]["b"]), cv(p["nb1_l1"]["b"])),
        cc(cv(p["nb0_l2"]["b"] + p["nb1_l2"]["b"]), p["nb_an1"]["g"]),
        cc(p["nb_an1"]["b"], cv(p["nb_al"]["b"])),
        cc(p["nb_an2"]["g"], p["nb_an2"]["b"]),
        cc(p["lin_k"]["b"], p["lin_v"]["b"]),
    ])
    ea4 = jnp.tile(edge_attr, (1, 2))
    c01, cae = _run_p3(gr, gqb, gs, ea4, sw, wcat, wal_c, wkv,
                       vec_3)

    z32 = jnp.zeros((N, 32), F32)
    z16 = jnp.zeros((N, 16), F32)
    uacc = _run_scat_wv(c01, dst, z32)
    dacc = _run_scat_ae(cae, dst, z16)

    zz = jnp.zeros((EMBED, EMBED), F32)
    w5 = jnp.block([[p["lin_ih"]["W"], zz],
                    [p["lin_hh"]["W"], p["lin_self"]["W"]]])
    vec_5 = jnp.stack([
        jnp.concatenate([p["lin_ih"]["b"] + p["lin_hh"]["b"],
                         p["lin_self"]["b"],
                         jnp.zeros((2 * EMBED,), F32)]),
        jnp.concatenate([p["out_proj"]["b"], p["norm2"]["g"],
                         p["norm2"]["b"], p["mlp_l2"]["b"]]),
        p["mlp_l1"]["b"],
    ])
    return _run_p5(uacc[0], uacc[1], dacc[0], dacc[1], h, center,
                   w5, p["out_proj"]["W"], p["mlp_l1"]["W"], p["mlp_l2"]["W"],
                   vec_5)


# speedup vs baseline: 1.6409x; 1.6409x over previous
import functools
import math

import jax
import jax.numpy as jnp
from jax import lax
from jax.experimental import pallas as pl
from jax.experimental.pallas import tpu as pltpu
from jax.experimental.pallas import tpu_sc as plsc

N = 50000
E = 800000
EMBED = 64
HEADS = 8
DH = EMBED // HEADS

NB = 2000
EB = 6400
XROW = 16

F32 = jnp.float32
_PREC = lax.Precision.DEFAULT


def _dot(a, b):
    return jax.lax.dot_general(a, b, (((1,), (0,)), ((), ())),
                               precision=_PREC, preferred_element_type=F32)


def _jmat(width, seg):
    r = lax.broadcasted_iota(jnp.int32, (width, width), 0) // seg
    c = lax.broadcasted_iota(jnp.int32, (width, width), 1) // seg
    return (r == c).astype(F32) * (1.0 / seg)


def _ln_mxu(x, g, b, jm, eps=1e-5):
    c0 = x - _dot(x, jm)
    v = _dot(c0 * c0, jm)
    return c0 * lax.rsqrt(v + eps) * g + b


def _groups_mats(width, seg):
    p = width // seg
    r = lax.broadcasted_iota(jnp.int32, (width, p), 0) // seg
    c = lax.broadcasted_iota(jnp.int32, (width, p), 1)
    avg = (r == c).astype(F32) * (1.0 / seg)
    return avg, (avg * seg).T


def _lnc(xc, g, b, width, seg, eps=1e-5):
    avg, exp_m = _groups_mats(width, seg)
    v = _dot(xc * xc, avg)
    return xc * _dot(lax.rsqrt(v + eps), exp_m) * g + b


def _head_mat():
    r = lax.broadcasted_iota(jnp.int32, (EMBED, HEADS), 0) // DH
    c = lax.broadcasted_iota(jnp.int32, (EMBED, HEADS), 1)
    return (r == c).astype(F32)


def _p1_body(x_ref, rot_ref, bosf_ref, bosrow_ref, w2_ref, w64_ref, vec_ref,
             center_ref, h_ref, q_ref):
    x0 = x_ref[:, 0:1]
    x1 = x_ref[:, 1:2]
    rot = rot_ref[...]
    r00 = rot[:, 0:1]
    r01 = rot[:, 1:2]
    r10 = rot[:, 2:3]
    r11 = rot[:, 3:4]
    rx0 = x0 * r00 + x1 * r10
    rx1 = x0 * r01 + x1 * r11
    w2 = w2_ref[...]
    j1 = _jmat(EMBED, EMBED)
    h1 = rx0 * w2[0:1, :] + rx1 * w2[1:2, :] + vec_ref[0]
    h1 = jnp.maximum(_ln_mxu(h1, vec_ref[1], vec_ref[2], j1), 0.0)
    h2 = _dot(h1, w64_ref[0]) + vec_ref[3]
    h2 = jnp.maximum(_ln_mxu(h2, vec_ref[4], vec_ref[5], j1), 0.0)
    c0 = _ln_mxu(_dot(h2, w64_ref[1]) + vec_ref[6], vec_ref[7], vec_ref[8], j1)
    bosf = bosf_ref[:, 0:1]
    center = c0 * (1.0 - bosf) + bosrow_ref[...] * bosf
    h = _ln_mxu(center, vec_ref[9], vec_ref[10], j1)
    q = _dot(h, w64_ref[2]) + vec_ref[11]
    center_ref[...] = center
    h_ref[...] = h
    q_ref[...] = q


def _run_p1(x, rot4, bosf, bosrow, w2, w64, vec):
    grid = (N // NB,)
    return pl.pallas_call(
        _p1_body,
        grid=grid,
        in_specs=[
            pl.BlockSpec((NB, 2), lambda i: (i, 0)),
            pl.BlockSpec((NB, 4), lambda i: (i, 0)),
            pl.BlockSpec((NB, 1), lambda i: (i, 0)),
            pl.BlockSpec((1, EMBED), lambda i: (0, 0)),
            pl.BlockSpec((2, EMBED), lambda i: (0, 0)),
            pl.BlockSpec((3, EMBED, EMBED), lambda i: (0, 0, 0)),
            pl.BlockSpec((12, EMBED), lambda i: (0, 0)),
        ],
        out_specs=[
            pl.BlockSpec((NB, EMBED), lambda i: (i, 0)),
            pl.BlockSpec((NB, EMBED), lambda i: (i, 0)),
            pl.BlockSpec((NB, EMBED), lambda i: (i, 0)),
        ],
        out_shape=[
            jax.ShapeDtypeStruct((N, EMBED), F32),
            jax.ShapeDtypeStruct((N, EMBED), F32),
            jax.ShapeDtypeStruct((N, EMBED), F32),
        ],
    )(x, rot4, bosf, bosrow, w2, w64, vec)


_NSC = 2
_NSUB = 16


def _sc_mesh():
    return plsc.VectorSubcoreMesh(core_axis_name="c", subcore_axis_name="s",
                                  num_cores=_NSC, num_subcores=_NSUB)


_SC_PARAMS = pltpu.CompilerParams(use_tc_tiling_on_sc=False)
_NW = _NSC * _NSUB
_G_CH = 200
_G_EPW = E // _NW
_G_NIT = _G_EPW // _G_CH


def _gather_body(t_hbm, tx_hbm, dst_hbm, src_hbm, gd_hbm,
                 idxd_v, idxs_v, bufd, bufs, semd, sems):
    c = lax.axis_index("c")
    s = lax.axis_index("s")
    base = (c * _NSUB + s) * _G_EPW

    def body(i, carry):
        e0 = base + i * _G_CH
        pltpu.sync_copy(dst_hbm.at[pl.ds(e0, _G_CH)], idxd_v)
        pltpu.sync_copy(src_hbm.at[pl.ds(e0, _G_CH)], idxs_v)
        cd = pltpu.async_copy(t_hbm.at[idxd_v], bufd, semd)
        cs = pltpu.async_copy(tx_hbm.at[idxs_v], bufs, sems)
        cd.wait()
        cs.wait()

        def rotmul(j, carry2):
            bufd[j, pl.ds(80, 16)] = (bufs[j, pl.ds(0, 16)] *
                                      bufd[j, pl.ds(64, 16)])
            return carry2

        lax.fori_loop(0, _G_CH, rotmul, 0)
        pltpu.sync_copy(bufd, gd_hbm.at[pl.ds(e0, _G_CH)])
        return carry

    lax.fori_loop(0, _G_NIT, body, 0)


def _run_gather(t_tab, tx_tab, dst, src):
    f = functools.partial(
        pl.kernel,
        out_type=jax.ShapeDtypeStruct((E, 128), F32),
        mesh=_sc_mesh(),
        compiler_params=_SC_PARAMS,
        scratch_types=[
            pltpu.VMEM((_G_CH,), jnp.int32),
            pltpu.VMEM((_G_CH,), jnp.int32),
            pltpu.VMEM((_G_CH, 128), F32),
            pltpu.VMEM((_G_CH, XROW), F32),
            pltpu.SemaphoreType.DMA,
            pltpu.SemaphoreType.DMA,
        ],
    )(_gather_body)
    return f(t_tab, tx_tab, dst, src)


def _p3_body(gd_ref, ea_ref, w2_ref, wcat_ref, wal_ref, wkv_ref,
             vec_ref, cwv_ref, cae_ref):
    gd = gd_ref[...]
    q = gd[:, 0:EMBED]
    prod = jnp.concatenate(
        [gd[:, 80:84], ea_ref[...] * gd[:, 64:68]], axis=1)
    catc = _dot(prod, w2_ref[...]) + vec_ref[2]
    cat = jnp.maximum(_lnc(catc, vec_ref[0], vec_ref[1], 2 * EMBED, EMBED),
                      0.0)
    smc = _dot(cat, wcat_ref[...]) + vec_ref[3, 0:EMBED]
    t1 = jnp.maximum(
        _lnc(smc, vec_ref[3, EMBED:], vec_ref[4, 0:EMBED], EMBED, EMBED), 0.0)
    alc = _dot(t1, wal_ref[...]) + vec_ref[4, EMBED:]
    nbr = _lnc(alc, vec_ref[5, 0:EMBED], vec_ref[5, EMBED:], EMBED, EMBED)
    kv = _dot(nbr, wkv_ref[...]) + vec_ref[6]
    k = kv[:, 0:EMBED]
    v = kv[:, EMBED:]
    hm = _head_mat()
    alpha = _dot(q * k, hm) * (1.0 / math.sqrt(float(DH)))
    ae = jnp.exp(alpha)
    wv = v * _dot(ae, hm.T)
    cwv_ref[...] = jnp.concatenate([wv[0:EB // 2], wv[EB // 2:]], axis=1)
    aep = jnp.concatenate([ae, jnp.zeros((EB, 8), F32)], axis=1)
    k8 = EB // 8
    cae_ref[...] = jnp.concatenate(
        [aep[k * k8:(k + 1) * k8] for k in range(8)], axis=1)


def _run_p3(gd, ea, w2, wcat, wal, wkv, vec):
    grid = (E // EB,)
    return pl.pallas_call(
        _p3_body,
        grid=grid,
        in_specs=[
            pl.BlockSpec((EB, 128), lambda i: (i, 0)),
            pl.BlockSpec((EB, 4), lambda i: (i, 0)),
            pl.BlockSpec((8, 2 * EMBED), lambda i: (0, 0)),
            pl.BlockSpec((2 * EMBED, EMBED), lambda i: (0, 0)),
            pl.BlockSpec((EMBED, EMBED), lambda i: (0, 0)),
            pl.BlockSpec((EMBED, 2 * EMBED), lambda i: (0, 0)),
            pl.BlockSpec((7, 2 * EMBED), lambda i: (0, 0)),
        ],
        out_specs=[
            pl.BlockSpec((EB // 2, 128), lambda i: (i, 0)),
            pl.BlockSpec((EB // 8, 128), lambda i: (i, 0)),
        ],
        out_shape=[
            jax.ShapeDtypeStruct((E // 2, 128), F32),
            jax.ShapeDtypeStruct((E // 8, 128), F32),
        ],
    )(gd, ea, w2, wcat, wal, wkv, vec)


_S_CH = 400
_S_EPW = E // _NSUB
_S_NIT = _S_EPW // _S_CH
_NPS = N // _NSUB


def _scat_wv_body(cwv_hbm, eo_hbm, z_hbm, out_hbm, idx_v, buf, acc, sem):
    c = lax.axis_index("c")
    s = lax.axis_index("s")
    pltpu.sync_copy(z_hbm.at[pl.ds(s * _NPS, _NPS)],
                    acc.at[pl.ds(s * _NPS, _NPS)])
    plsc.subcore_barrier()
    base_r = s * (_S_EPW // 2)
    half = _S_CH // 2

    def body(i, carry):
        r0 = base_r + i * half
        pltpu.sync_copy(eo_hbm.at[pl.ds(r0, half)], idx_v)
        pltpu.sync_copy(cwv_hbm.at[pl.ds(r0, half), pl.ds(32 * c, 32)], buf)
        pltpu.sync_copy(buf, acc.at[idx_v], add=True)
        pltpu.sync_copy(eo_hbm.at[pl.ds(E // 2 + r0, half)], idx_v)
        pltpu.sync_copy(cwv_hbm.at[pl.ds(r0, half), pl.ds(64 + 32 * c, 32)],
                        buf)
        pltpu.sync_copy(buf, acc.at[idx_v], add=True)
        return carry

    lax.fori_loop(0, _S_NIT, body, 0)
    plsc.subcore_barrier()
    pltpu.sync_copy(acc.at[pl.ds(s * _NPS, _NPS)],
                    out_hbm.at[c, pl.ds(s * _NPS, _NPS), pl.ds(0, 32)])


def _run_scat_wv(cwv, dst_eo, z32):
    f = functools.partial(
        pl.kernel,
        out_type=jax.ShapeDtypeStruct((2, N, 128), F32),
        mesh=_sc_mesh(),
        compiler_params=_SC_PARAMS,
        scratch_types=[
            pltpu.VMEM((_S_CH // 2,), jnp.int32),
            pltpu.VMEM((_S_CH // 2, 32), F32),
            pltpu.VMEM_SHARED((N, 32), F32),
            pltpu.SemaphoreType.DMA,
        ],
    )(_scat_wv_body)
    return f(cwv, dst_eo, z32)


_A_RPS = 3200
_A_RCH = 200
_A_RPC = (E // 8) // _NSC


def _scat_ae_body(cae_hbm, p8_hbm, z_hbm, out_hbm, idx_v, buf, acc, sem):
    c = lax.axis_index("c")
    s = lax.axis_index("s")
    pltpu.sync_copy(z_hbm.at[pl.ds(s * _NPS, _NPS)],
                    acc.at[pl.ds(s * _NPS, _NPS)])
    plsc.subcore_barrier()

    def body(i, carry):
        r_loc = s * _A_RPS + i * _A_RCH

        @pl.when(r_loc < _A_RPC)
        def _():
            r_glob = c * _A_RPC + r_loc

            def phase(k, carry2):
                off = k * (E // 8) + r_glob
                pltpu.sync_copy(p8_hbm.at[pl.ds(off, _A_RCH)], idx_v)
                pltpu.sync_copy(
                    cae_hbm.at[pl.ds(r_glob, _A_RCH), pl.ds(16 * k, 16)],
                    buf)
                pltpu.sync_copy(buf, acc.at[idx_v], add=True)
                return carry2

            lax.fori_loop(0, 8, phase, 0)

        return carry

    lax.fori_loop(0, _A_RPS // _A_RCH, body, 0)
    plsc.subcore_barrier()
    pltpu.sync_copy(acc.at[pl.ds(s * _NPS, _NPS)],
                    out_hbm.at[c, pl.ds(s * _NPS, _NPS), pl.ds(0, 16)])


def _run_scat_ae(cae, dst_p8, z16):
    f = functools.partial(
        pl.kernel,
        out_type=jax.ShapeDtypeStruct((2, N, 128), F32),
        mesh=_sc_mesh(),
        compiler_params=_SC_PARAMS,
        scratch_types=[
            pltpu.VMEM((_A_RCH,), jnp.int32),
            pltpu.VMEM((_A_RCH, 16), F32),
            pltpu.VMEM_SHARED((N, 16), F32),
            pltpu.SemaphoreType.DMA,
        ],
    )(_scat_ae_body)
    return f(cae, dst_p8, z16)


def _p5_body(u0_ref, u1_ref, d0_ref, d1_ref, h_ref, c_ref,
             w5_ref, wout_ref, wm1_ref, wm2_ref, vec_ref, out_ref):
    dsum = d0_ref[0][:, 0:HEADS] + d1_ref[0][:, 0:HEADS]
    u = jnp.concatenate([u0_ref[0][:, 0:32], u1_ref[0][:, 0:32]], axis=1)
    hm = _head_mat()
    agg = u * _dot(1.0 / (dsum + 1e-16), hm.T)
    h = h_ref[...]
    cath = jnp.concatenate([agg, h], axis=1)
    gs = _dot(cath, w5_ref[...]) + vec_ref[0, 0:2 * EMBED]
    gate = jax.nn.sigmoid(gs[:, 0:EMBED])
    selfh = gs[:, EMBED:]
    upd = agg + gate * (selfh - agg)
    c2 = c_ref[...] + _dot(upd, wout_ref[...]) + vec_ref[1, 0:EMBED]
    j1 = _jmat(EMBED, EMBED)
    h2 = _ln_mxu(c2, vec_ref[1, EMBED:2 * EMBED],
                 vec_ref[1, 2 * EMBED:3 * EMBED], j1)
    ff = _dot(jnp.maximum(_dot(h2, wm1_ref[...]) + vec_ref[2], 0.0),
              wm2_ref[...]) + vec_ref[1, 3 * EMBED:]
    out_ref[...] = c2 + ff


def _run_p5(uacc, dacc, h, center, w5, wout, wm1, wm2, vec):
    grid = (N // NB,)
    return pl.pallas_call(
        _p5_body,
        grid=grid,
        in_specs=[
            pl.BlockSpec((1, NB, 128), lambda i: (0, i, 0)),
            pl.BlockSpec((1, NB, 128), lambda i: (1, i, 0)),
            pl.BlockSpec((1, NB, 128), lambda i: (0, i, 0)),
            pl.BlockSpec((1, NB, 128), lambda i: (1, i, 0)),
            pl.BlockSpec((NB, EMBED), lambda i: (i, 0)),
            pl.BlockSpec((NB, EMBED), lambda i: (i, 0)),
            pl.BlockSpec((2 * EMBED, 2 * EMBED), lambda i: (0, 0)),
            pl.BlockSpec((EMBED, EMBED), lambda i: (0, 0)),
            pl.BlockSpec((EMBED, 4 * EMBED), lambda i: (0, 0)),
            pl.BlockSpec((4 * EMBED, EMBED), lambda i: (0, 0)),
            pl.BlockSpec((3, 4 * EMBED), lambda i: (0, 0)),
        ],
        out_specs=pl.BlockSpec((NB, EMBED), lambda i: (i, 0)),
        out_shape=jax.ShapeDtypeStruct((N, EMBED), F32),
    )(uacc, uacc, dacc, dacc, h, center, w5, wout, wm1, wm2, vec)


def kernel(x, t, edge_index, edge_attr, bos_mask, rotate_mat, params):
    p = params
    src = edge_index[0]
    dst = edge_index[1]
    rot4 = rotate_mat.reshape(N, 4)
    bosf = bos_mask.astype(F32).reshape(N, 1)
    bosrow = p["bos_token"][t].reshape(1, EMBED)

    w2_1 = p["ce_l1"]["W"]
    w64_1 = jnp.stack([p["ce_l2"]["W"], p["ce_l3"]["W"], p["lin_q"]["W"]])
    vec_1 = jnp.stack([
        p["ce_l1"]["b"], p["ce_n1"]["g"], p["ce_n1"]["b"],
        p["ce_l2"]["b"], p["ce_n2"]["g"], p["ce_n2"]["b"],
        p["ce_l3"]["b"], p["ce_n3"]["g"], p["ce_n3"]["b"],
        p["norm1"]["g"], p["norm1"]["b"], p["lin_q"]["b"],
    ])
    center, h, q_nodes = _run_p1(x, rot4, bosf, bosrow, w2_1, w64_1, vec_1)

    rdup = jnp.stack([rot4[:, 0], rot4[:, 2], rot4[:, 1], rot4[:, 3]], axis=1)
    t_tab = jnp.concatenate(
        [q_nodes, rdup, rdup, jnp.zeros((N, 56), F32)], axis=1)
    tx_tab = jnp.concatenate([x, x, jnp.zeros((N, XROW - 4), F32)], axis=1)
    gd = _run_gather(t_tab, tx_tab, dst, src)

    cmat = jnp.eye(EMBED, dtype=F32) - jnp.full((EMBED, EMBED), 1.0 / EMBED,
                                                F32)
    ctr = lambda w: w @ cmat
    wa = p["nb0_l1"]["W"]
    wb = p["nb1_l1"]["W"]
    z64 = jnp.zeros((EMBED,), F32)
    sw = jnp.stack([
        jnp.concatenate([wa[0], z64]), jnp.concatenate([wa[0], z64]),
        jnp.concatenate([wa[1], z64]), jnp.concatenate([wa[1], z64]),
        jnp.concatenate([z64, wb[0]]), jnp.concatenate([z64, wb[0]]),
        jnp.concatenate([z64, wb[1]]), jnp.concatenate([z64, wb[1]]),
    ])
    sw = jnp.concatenate([sw[:, 0:EMBED] @ cmat, sw[:, EMBED:] @ cmat],
                         axis=1)
    wcat = ctr(jnp.concatenate([p["nb0_l2"]["W"], p["nb1_l2"]["W"]], axis=0))
    wal_c = ctr(p["nb_al"]["W"])
    wkv = jnp.concatenate([p["lin_k"]["W"], p["lin_v"]["W"]], axis=1)
    cc = lambda a, b: jnp.concatenate([a, b])
    cv = lambda v: v - jnp.mean(v)
    vec_3 = jnp.stack([
        cc(p["nb0_n1"]["g"], p["nb1_n1"]["g"]),
        cc(p["nb0_n1"]["b"], p["nb1_n1"]["b"]),
        cc(cv(p["nb0_l1"]["b"]), cv(p["nb1_l1"]["b"])),
        cc(cv(p["nb0_l2"]["b"] + p["nb1_l2"]["b"]), p["nb_an1"]["g"]),
        cc(p["nb_an1"]["b"], cv(p["nb_al"]["b"])),
        cc(p["nb_an2"]["g"], p["nb_an2"]["b"]),
        cc(p["lin_k"]["b"], p["lin_v"]["b"]),
    ])
    ea4 = jnp.tile(edge_attr, (1, 2))
    cwv, cae = _run_p3(gd, ea4, sw, wcat, wal_c, wkv, vec_3)

    dst_eo = jnp.swapaxes(dst.reshape(E // EB, 2, EB // 2), 0, 1).reshape(E)
    dst_p8 = jnp.swapaxes(dst.reshape(E // EB, 8, EB // 8), 0, 1).reshape(E)
    z32 = jnp.zeros((N, 32), F32)
    z16 = jnp.zeros((N, 16), F32)
    uacc = _run_scat_wv(cwv, dst_eo, z32)
    dacc = _run_scat_ae(cae, dst_p8, z16)

    w64_5 = jnp.block([[p["lin_ih"]["W"], jnp.zeros((EMBED, EMBED), F32)],
                       [p["lin_hh"]["W"], p["lin_self"]["W"]]])
    vec_5 = jnp.stack([
        jnp.concatenate([p["lin_ih"]["b"] + p["lin_hh"]["b"],
                         p["lin_self"]["b"],
                         jnp.zeros((2 * EMBED,), F32)]),
        jnp.concatenate([p["out_proj"]["b"], p["norm2"]["g"],
                         p["norm2"]["b"], p["mlp_l2"]["b"]]),
        p["mlp_l1"]["b"],
    ])
    return _run_p5(uacc, dacc, h, center,
                   w64_5, p["out_proj"]["W"], p["mlp_l1"]["W"],
                   p["mlp_l2"]["W"], vec_5)
